# baseline (speedup 1.0000x reference)
import functools

import jax
import jax.numpy as jnp
from jax import lax
from jax.experimental import pallas as pl
from jax.experimental.pallas import tpu as pltpu
from jax.experimental.pallas import tpu_sc as plsc

VOCAB_SIZE = 30522
MASK_SELECTION_RATE = 0.15
MASK_TOKEN_ID = 103
L = 320
B = 16
S = 2048
LANES = 16
CHUNKS = S // LANES
NC = 2
NS = 16


def _sc_body(inputs_hbm, code_hbm, tok_out, pos_out, ids_out, w_out,
             inp_v, code_v, tok_v, pos_v, ids_v, w_v):
    s = lax.axis_index("s")
    row = s

    @pl.when(row < B)
    def _():
        pltpu.sync_copy(inputs_hbm.at[row], inp_v)
        pltpu.sync_copy(code_hbm.at[row], code_v)

        zero_i = jnp.zeros((LANES,), jnp.int32)
        zero_f = jnp.zeros((LANES,), jnp.float32)
        for i in range(L // LANES):
            pos_v[pl.ds(i * LANES, LANES)] = zero_i
            ids_v[pl.ds(i * LANES, LANES)] = zero_i
            w_v[pl.ds(i * LANES, LANES)] = zero_f

        iota = lax.iota(jnp.int32, LANES)
        ones_f = jnp.ones((LANES,), jnp.float32)

        def body(i, carry):
            off = i * LANES
            tok = inp_v[pl.ds(off, LANES)]
            cd = code_v[pl.ds(off, LANES)]
            sel_raw = (cd != -1) & (tok != 0)
            inc = jnp.where(sel_raw, 1, 0).astype(jnp.int32)
            cs = plsc.cumsum(inc) + carry
            selected = sel_raw & (cs <= L)
            tok_v[pl.ds(off, LANES)] = jnp.where(selected & (cd >= 0), cd, tok)
            rank = cs - 1
            plsc.store_scatter(pos_v, [rank], off + iota, mask=selected)
            plsc.store_scatter(ids_v, [rank], tok, mask=selected)
            plsc.store_scatter(w_v, [rank], ones_f, mask=selected)
            return cs[15]

        lax.fori_loop(0, CHUNKS, body, jnp.int32(0))

        pltpu.sync_copy(tok_v, tok_out.at[row])
        pltpu.sync_copy(pos_v, pos_out.at[row])
        pltpu.sync_copy(ids_v, ids_out.at[row])
        pltpu.sync_copy(w_v, w_out.at[row])


@functools.lru_cache(maxsize=1)
def _build_sc_call():
    mesh = plsc.VectorSubcoreMesh(
        core_axis_name="c", subcore_axis_name="s",
        num_cores=1, num_subcores=NS)
    return pl.kernel(
        _sc_body,
        out_type=(
            jax.ShapeDtypeStruct((B, S), jnp.int32),
            jax.ShapeDtypeStruct((B, L), jnp.int32),
            jax.ShapeDtypeStruct((B, L), jnp.int32),
            jax.ShapeDtypeStruct((B, L), jnp.float32),
        ),
        mesh=mesh,
        compiler_params=pltpu.CompilerParams(needs_layout_passes=False),
        scratch_types=[
            pltpu.VMEM((S,), jnp.int32),
            pltpu.VMEM((S,), jnp.int32),
            pltpu.VMEM((S,), jnp.int32),
            pltpu.VMEM((L,), jnp.int32),
            pltpu.VMEM((L,), jnp.int32),
            pltpu.VMEM((L,), jnp.float32),
        ],
    )


@functools.lru_cache(maxsize=1)
def _code_const():
    key = jax.random.key(42)
    k_sel, k_act, k_rand = jax.random.split(key, 3)
    u = jax.random.uniform(k_sel, (B, S))
    r = jax.random.uniform(k_act, (B, S))
    rand_tok = jax.random.randint(k_rand, (B, S), 0, VOCAB_SIZE, dtype=jnp.int32)
    code = jnp.where(
        u < MASK_SELECTION_RATE,
        jnp.where(r < 0.8, MASK_TOKEN_ID, jnp.where(r < (0.8 + 0.1), rand_tok, -2)),
        -1,
    ).astype(jnp.int32)
    return jax.device_get(code)


def kernel(inputs):
    code = jnp.asarray(_code_const())
    return _build_sc_call()(inputs, code)


# speedup vs baseline: 1.7297x; 1.7297x over previous
import functools

import jax
import jax.numpy as jnp
from jax import lax
from jax.experimental import pallas as pl
from jax.experimental.pallas import tpu as pltpu
from jax.experimental.pallas import tpu_sc as plsc

VOCAB_SIZE = 30522
MASK_SELECTION_RATE = 0.15
MASK_TOKEN_ID = 103
L = 320
B = 16
S = 2048
LANES = 16
CHUNKS = S // LANES
NC = 2
NS = 16


def _sc_body(inputs_hbm, code_hbm, tok_out, pos_out, ids_out, w_out,
             inp_v, code_v, tok_v, pos_v, ids_v, w_v):
    s = lax.axis_index("s")
    row = s

    @pl.when(row < B)
    def _():
        pltpu.sync_copy(inputs_hbm.at[row], inp_v)
        pltpu.sync_copy(code_hbm.at[row], code_v)

        zero_i = jnp.zeros((LANES,), jnp.int32)
        zero_f = jnp.zeros((LANES,), jnp.float32)
        for i in range(L // LANES):
            pos_v[pl.ds(i * LANES, LANES)] = zero_i
            ids_v[pl.ds(i * LANES, LANES)] = zero_i
            w_v[pl.ds(i * LANES, LANES)] = zero_f

        iota = lax.iota(jnp.int32, LANES)
        ones_f = jnp.ones((LANES,), jnp.float32)

        def body(i, carry):
            off = i * LANES
            tok = inp_v[pl.ds(off, LANES)]
            cd = code_v[pl.ds(off, LANES)]
            sel_raw = (cd != -1) & (tok != 0)
            inc = jnp.where(sel_raw, 1, 0).astype(jnp.int32)
            cs = plsc.cumsum(inc) + carry
            selected = sel_raw & (cs <= L)
            tok_v[pl.ds(off, LANES)] = jnp.where(selected & (cd >= 0), cd, tok)
            rank = cs - 1
            plsc.store_scatter(pos_v, [rank], off + iota, mask=selected)
            plsc.store_scatter(ids_v, [rank], tok, mask=selected)
            plsc.store_scatter(w_v, [rank], ones_f, mask=selected)
            return cs[15]

        lax.fori_loop(0, CHUNKS, body, jnp.int32(0))

        pltpu.sync_copy(tok_v, tok_out.at[row])
        pltpu.sync_copy(pos_v, pos_out.at[row])
        pltpu.sync_copy(ids_v, ids_out.at[row])
        pltpu.sync_copy(w_v, w_out.at[row])


@functools.lru_cache(maxsize=1)
def _build_sc_call():
    mesh = plsc.VectorSubcoreMesh(
        core_axis_name="c", subcore_axis_name="s",
        num_cores=1, num_subcores=NS)
    return pl.kernel(
        _sc_body,
        out_type=(
            jax.ShapeDtypeStruct((B, S), jnp.int32),
            jax.ShapeDtypeStruct((B, L), jnp.int32),
            jax.ShapeDtypeStruct((B, L), jnp.int32),
            jax.ShapeDtypeStruct((B, L), jnp.float32),
        ),
        mesh=mesh,
        compiler_params=pltpu.CompilerParams(needs_layout_passes=False),
        scratch_types=[
            pltpu.VMEM((S,), jnp.int32),
            pltpu.VMEM((S,), jnp.int32),
            pltpu.VMEM((S,), jnp.int32),
            pltpu.VMEM((L,), jnp.int32),
            pltpu.VMEM((L,), jnp.int32),
            pltpu.VMEM((L,), jnp.float32),
        ],
    )


@functools.lru_cache(maxsize=1)
def _code_const():
    with jax.ensure_compile_time_eval():
        return _code_eager()


def _code_eager():
    key = jax.random.key(42)
    k_sel, k_act, k_rand = jax.random.split(key, 3)
    u = jax.random.uniform(k_sel, (B, S))
    r = jax.random.uniform(k_act, (B, S))
    rand_tok = jax.random.randint(k_rand, (B, S), 0, VOCAB_SIZE, dtype=jnp.int32)
    code = jnp.where(
        u < MASK_SELECTION_RATE,
        jnp.where(r < 0.8, MASK_TOKEN_ID, jnp.where(r < (0.8 + 0.1), rand_tok, -2)),
        -1,
    ).astype(jnp.int32)
    return jax.device_get(code)


def kernel(inputs):
    code = jnp.asarray(_code_const())
    return _build_sc_call()(inputs, code)


# speedup vs baseline: 1.7898x; 1.0348x over previous
import functools

import jax
import jax.numpy as jnp
from jax import lax
from jax.experimental import pallas as pl
from jax.experimental.pallas import tpu as pltpu
from jax.experimental.pallas import tpu_sc as plsc

VOCAB_SIZE = 30522
MASK_SELECTION_RATE = 0.15
MASK_TOKEN_ID = 103
L = 320
B = 16
S = 2048
LANES = 16
CHUNKS = S // LANES
NS = 16


def _sc_body(inputs_hbm, code_hbm, tok_out, pos_out, ids_out, w_out,
             inp_v, code_v, tok_v, pos_v, ids_v, w_v, sem_a, sem_b):
    row = lax.axis_index("s")

    cp_a = pltpu.async_copy(inputs_hbm.at[row], inp_v, sem_a)
    cp_b = pltpu.async_copy(code_hbm.at[row], code_v, sem_b)
    cp_a.wait()
    cp_b.wait()

    iota = lax.iota(jnp.int32, LANES)

    def body(i, carry_vec):
        off = i * LANES
        tok = inp_v[pl.ds(off, LANES)]
        cd = code_v[pl.ds(off, LANES)]
        sel_raw = (cd != -1) & (tok != 0)
        inc = jnp.where(sel_raw, 1, 0).astype(jnp.int32)
        cs = plsc.cumsum(inc) + carry_vec
        selected = sel_raw & (cs <= L)
        tok_v[pl.ds(off, LANES)] = jnp.where(selected & (cd >= 0), cd, tok)
        plsc.store_scatter(pos_v, [cs - 1], off + iota, mask=selected)
        return carry_vec + plsc.all_reduce_population_count(sel_raw)

    total = lax.fori_loop(0, CHUNKS, body, jnp.zeros((LANES,), jnp.int32))
    n_sel = jnp.minimum(total, L)

    ones_f = jnp.ones((LANES,), jnp.float32)
    zero_f = jnp.zeros((LANES,), jnp.float32)
    for i in range(L // LANES):
        sl = pl.ds(i * LANES, LANES)
        valid = (i * LANES + iota) < n_sel
        p = jnp.where(valid, pos_v[sl], 0)
        pos_v[sl] = p
        g = plsc.load_gather(inp_v, [p])
        ids_v[sl] = jnp.where(valid, g, 0)
        w_v[sl] = jnp.where(valid, ones_f, zero_f)

    st_a = pltpu.async_copy(tok_v, tok_out.at[row], sem_a)
    st_b = pltpu.async_copy(pos_v, pos_out.at[row], sem_b)
    st_a.wait()
    st_b.wait()
    st_c = pltpu.async_copy(ids_v, ids_out.at[row], sem_a)
    st_d = pltpu.async_copy(w_v, w_out.at[row], sem_b)
    st_c.wait()
    st_d.wait()


@functools.lru_cache(maxsize=1)
def _build_sc_call():
    mesh = plsc.VectorSubcoreMesh(
        core_axis_name="c", subcore_axis_name="s",
        num_cores=1, num_subcores=NS)
    return pl.kernel(
        _sc_body,
        out_type=(
            jax.ShapeDtypeStruct((B, S), jnp.int32),
            jax.ShapeDtypeStruct((B, L), jnp.int32),
            jax.ShapeDtypeStruct((B, L), jnp.int32),
            jax.ShapeDtypeStruct((B, L), jnp.float32),
        ),
        mesh=mesh,
        compiler_params=pltpu.CompilerParams(needs_layout_passes=False),
        scratch_types=[
            pltpu.VMEM((S,), jnp.int32),
            pltpu.VMEM((S,), jnp.int32),
            pltpu.VMEM((S,), jnp.int32),
            pltpu.VMEM((L,), jnp.int32),
            pltpu.VMEM((L,), jnp.int32),
            pltpu.VMEM((L,), jnp.float32),
            pltpu.SemaphoreType.DMA,
            pltpu.SemaphoreType.DMA,
        ],
    )


@functools.lru_cache(maxsize=1)
def _code_const():
    with jax.ensure_compile_time_eval():
        key = jax.random.key(42)
        k_sel, k_act, k_rand = jax.random.split(key, 3)
        u = jax.random.uniform(k_sel, (B, S))
        r = jax.random.uniform(k_act, (B, S))
        rand_tok = jax.random.randint(k_rand, (B, S), 0, VOCAB_SIZE,
                                      dtype=jnp.int32)
        code = jnp.where(
            u < MASK_SELECTION_RATE,
            jnp.where(r < 0.8, MASK_TOKEN_ID,
                      jnp.where(r < (0.8 + 0.1), rand_tok, -2)),
            -1,
        ).astype(jnp.int32)
        return jax.device_get(code)


def kernel(inputs):
    code = jnp.asarray(_code_const())
    return _build_sc_call()(inputs, code)
